# baseline (speedup 1.0000x reference)
import jax
import jax.numpy as jnp
from jax import lax
from jax.experimental import pallas as pl
from jax.experimental.pallas import tpu as pltpu

N = 1048576
D = 64
SEL_ROWS = N // 128
BLKQ = 128


def _matvec_kernel(feaq_ref, wbig_ref, out_ref):
    out_ref[...] = jnp.dot(feaq_ref[...], wbig_ref[...],
                           preferred_element_type=jnp.float32)


def _matvec(feaq, wbig):
    return pl.pallas_call(
        _matvec_kernel,
        grid=(SEL_ROWS // BLKQ,),
        in_specs=[
            pl.BlockSpec((BLKQ, 8192), lambda i: (i, 0)),
            pl.BlockSpec((8192, 128), lambda i: (0, 0)),
        ],
        out_specs=pl.BlockSpec((BLKQ, 128), lambda i: (i, 0)),
        out_shape=jax.ShapeDtypeStruct((SEL_ROWS, 128), jnp.float32),
    )(feaq, wbig)


def _select_kernel(k_ref, pred_ref, pruned_ref, keep_ref):
    k = k_ref[0]
    pred = pred_ref[...]
    u = lax.bitcast_convert_type(pred, jnp.int32)
    key = jnp.where(u < 0, u ^ jnp.int32(0x7FFFFFFF), u)
    key = key ^ jnp.int32(-2147483648)

    c = jnp.sum((lax.shift_right_logical(key, 31) == 0).astype(jnp.int32))
    take1 = k > c
    rank = jnp.where(take1, k - c, k)
    prefix = jnp.where(take1, jnp.int32(1), jnp.int32(0))
    for b in range(30, -1, -1):
        match = lax.shift_right_logical(key, b + 1) == prefix
        bit0 = (lax.shift_right_logical(key, b) & 1) == 0
        c = jnp.sum((match & bit0).astype(jnp.int32))
        take1 = rank > c
        rank = jnp.where(take1, rank - c, rank)
        prefix = lax.shift_left(prefix, 1) | jnp.where(take1, jnp.int32(1),
                                                       jnp.int32(0))

    up = prefix ^ jnp.int32(-2147483648)
    up = jnp.where(up < 0, up ^ jnp.int32(0x7FFFFFFF), up)
    thres = lax.bitcast_convert_type(up, jnp.float32)

    keep = pred > thres
    pruned_ref[...] = pred * keep.astype(jnp.float32)
    keep_ref[...] = keep.astype(jnp.int8)


def _select(pred2d, k_arr):
    return pl.pallas_call(
        _select_kernel,
        in_specs=[
            pl.BlockSpec(memory_space=pltpu.SMEM),
            pl.BlockSpec(memory_space=pltpu.VMEM),
        ],
        out_shape=[
            jax.ShapeDtypeStruct(pred2d.shape, jnp.float32),
            jax.ShapeDtypeStruct(pred2d.shape, jnp.int8),
        ],
    )(k_arr, pred2d)


def kernel(fea, W_cls, target_points_num):
    k_arr = jnp.asarray(N - target_points_num, jnp.int32).reshape(1)
    wbig = jnp.kron(jnp.eye(128, dtype=jnp.float32), W_cls)
    feaq = fea.reshape(SEL_ROWS, 8192)
    pred2d = _matvec(feaq, wbig)
    pruned2d, keep2d = _select(pred2d, k_arr)
    pruned = pruned2d.reshape(N, 1)
    keep = keep2d.reshape(N).astype(jnp.bool_)
    return pruned, keep


# speedup vs baseline: 1.3492x; 1.3492x over previous
import jax
import jax.numpy as jnp
from jax import lax
from jax.experimental import pallas as pl
from jax.experimental.pallas import tpu as pltpu

N = 1048576
D = 64
SEL_ROWS = N // 128
BLK = 8192
STEPS = N // BLK
NBUF = 6
OUTR = BLK // 128


def _matvec_kernel(fea_ref, w_ref, out_ref, bufs, sems):
    i = pl.program_id(0)

    def start(j, slot):
        pltpu.make_async_copy(
            fea_ref.at[pl.ds(j * BLK, BLK), :], bufs.at[slot], sems.at[slot]
        ).start()

    @pl.when(i == 0)
    def _():
        for j in range(NBUF - 1):
            start(j, j)

    nxt = i + NBUF - 1

    @pl.when(nxt < STEPS)
    def _():
        start(nxt, lax.rem(nxt, NBUF))

    slot = lax.rem(i, NBUF)
    pltpu.make_async_copy(
        fea_ref.at[pl.ds(i * BLK, BLK), :], bufs.at[slot], sems.at[slot]
    ).wait()
    s = jnp.dot(bufs[slot], w_ref[...], preferred_element_type=jnp.float32)
    out_ref[...] = s.reshape(OUTR, 128)


def _matvec(fea, w):
    return pl.pallas_call(
        _matvec_kernel,
        grid=(STEPS,),
        in_specs=[
            pl.BlockSpec(memory_space=pl.ANY),
            pl.BlockSpec((D, 1), lambda i: (0, 0)),
        ],
        out_specs=pl.BlockSpec((OUTR, 128), lambda i: (i, 0)),
        out_shape=jax.ShapeDtypeStruct((SEL_ROWS, 128), jnp.float32),
        scratch_shapes=[
            pltpu.VMEM((NBUF, BLK, D), jnp.float32),
            pltpu.SemaphoreType.DMA((NBUF,)),
        ],
    )(fea, w)


def _select_kernel(k_ref, pred_ref, pruned_ref, keep_ref):
    k = k_ref[0]
    pred = pred_ref[...]
    u = lax.bitcast_convert_type(pred, jnp.int32)
    key = jnp.where(u < 0, u ^ jnp.int32(0x7FFFFFFF), u)
    key = key ^ jnp.int32(-2147483648)

    c = jnp.sum((lax.shift_right_logical(key, 31) == 0).astype(jnp.int32))
    take1 = k > c
    rank = jnp.where(take1, k - c, k)
    prefix = jnp.where(take1, jnp.int32(1), jnp.int32(0))
    for b in range(30, -1, -1):
        match = lax.shift_right_logical(key, b + 1) == prefix
        bit0 = (lax.shift_right_logical(key, b) & 1) == 0
        c = jnp.sum((match & bit0).astype(jnp.int32))
        take1 = rank > c
        rank = jnp.where(take1, rank - c, rank)
        prefix = lax.shift_left(prefix, 1) | jnp.where(take1, jnp.int32(1),
                                                       jnp.int32(0))

    up = prefix ^ jnp.int32(-2147483648)
    up = jnp.where(up < 0, up ^ jnp.int32(0x7FFFFFFF), up)
    thres = lax.bitcast_convert_type(up, jnp.float32)

    keep = pred > thres
    pruned_ref[...] = pred * keep.astype(jnp.float32)
    keep_ref[...] = keep.astype(jnp.int8)


def _select(pred2d, k_arr):
    return pl.pallas_call(
        _select_kernel,
        in_specs=[
            pl.BlockSpec(memory_space=pltpu.SMEM),
            pl.BlockSpec(memory_space=pltpu.VMEM),
        ],
        out_shape=[
            jax.ShapeDtypeStruct(pred2d.shape, jnp.float32),
            jax.ShapeDtypeStruct(pred2d.shape, jnp.int8),
        ],
    )(k_arr, pred2d)


def kernel(fea, W_cls, target_points_num):
    k_arr = jnp.asarray(N - target_points_num, jnp.int32).reshape(1)
    pred2d = _matvec(fea, W_cls)
    pruned2d, keep2d = _select(pred2d, k_arr)
    pruned = pruned2d.reshape(N, 1)
    keep = keep2d.reshape(N).astype(jnp.bool_)
    return pruned, keep
